# Initial kernel scaffold; baseline (speedup 1.0000x reference)
import jax
import jax.numpy as jnp
from jax.experimental import pallas as pl


def kernel(pred, target):
    raise NotImplementedError("write your pallas kernel here")


# speedup vs baseline: 2.5125x; 2.5125x over previous
import functools

import jax
import jax.numpy as jnp
import numpy as np
from jax import lax
from jax.experimental import pallas as pl
from jax.experimental.pallas import tpu as pltpu
from jax.experimental.pallas import tpu_sc as plsc

NC = 2
NS = 16
NW = NC * NS
L = 16
BINS = 10

B = 16
C = 3
P = 512 * 512
PW = P // NW
V = PW // L

_EDGES = np.arange(BINS + 1, dtype=np.float32) / np.float32(10)
_EDGES[BINS] += np.float32(1e-6)
EDGES = [float(e) for e in _EDGES]

_LOG_C = [float(np.float32(2.0 / k)) for k in (15, 13, 11, 9, 7, 5, 3, 1)]

_mesh = plsc.VectorSubcoreMesh(
    core_axis_name="c", subcore_axis_name="s", num_cores=NC, num_subcores=NS
)


@functools.partial(
    pl.kernel,
    out_type=jax.ShapeDtypeStruct((2 * NW * L,), jnp.float32),
    mesh=_mesh,
    compiler_params=pltpu.CompilerParams(needs_layout_passes=False),
    scratch_types=[
        pltpu.VMEM((PW,), jnp.float32),
        pltpu.VMEM((PW,), jnp.float32),
        pltpu.VMEM((PW,), jnp.float32),
        pltpu.VMEM((PW,), jnp.int32),
        pltpu.VMEM((L,), jnp.float32),
        pltpu.VMEM((L,), jnp.float32),
    ],
)
def _ghm_partials(pred_hbm, tgt_hbm, out_hbm, p0_v, p1_v, p2_v, t_v, cnt_v, s_v):
    wid = lax.axis_index("s") * NC + lax.axis_index("c")
    base = wid * jnp.int32(PW)
    cnt_v[...] = jnp.zeros((L,), jnp.float32)
    s_v[...] = jnp.zeros((L,), jnp.float32)
    ones = jnp.ones((L,), jnp.float32)

    def vec_body(j, carry):
        sl = pl.ds(j * jnp.int32(L), L)
        p0 = p0_v[sl]
        p1 = p1_v[sl]
        p2 = p2_v[sl]
        t = t_v[sl]
        m = jnp.maximum(jnp.maximum(p0, p1), p2)
        x0 = p0 - m
        x1 = p1 - m
        x2 = p2 - m
        e0 = jnp.exp(x0)
        e1 = jnp.exp(x1)
        e2 = jnp.exp(x2)
        s = e0 + e1 + e2
        is0 = t == 0
        is1 = t == 1
        xt = jnp.where(is0, x0, jnp.where(is1, x1, x2))
        et = jnp.where(is0, e0, jnp.where(is1, e1, e2))
        g = 1.0 - et / s
        z = (s - 1.0) / (s + 1.0)
        u = z * z
        poly = jnp.full((L,), _LOG_C[0], jnp.float32)
        for ck in _LOG_C[1:]:
            poly = poly * u + ck
        picked = xt - poly * z
        bin_ = jnp.zeros((L,), jnp.int32)
        onei = jnp.ones((L,), jnp.int32)
        for i in range(1, BINS + 1):
            bin_ = jnp.where(g >= EDGES[i], bin_ + onei, bin_)
        plsc.addupdate_scatter(cnt_v, [bin_], ones)
        plsc.addupdate_scatter(s_v, [bin_], picked)
        return carry

    i32 = jnp.int32
    for b in range(B):
        pltpu.sync_copy(pred_hbm.at[pl.ds(base + i32((b * C + 0) * P), PW)], p0_v)
        pltpu.sync_copy(pred_hbm.at[pl.ds(base + i32((b * C + 1) * P), PW)], p1_v)
        pltpu.sync_copy(pred_hbm.at[pl.ds(base + i32((b * C + 2) * P), PW)], p2_v)
        pltpu.sync_copy(tgt_hbm.at[pl.ds(base + i32(b * P), PW)], t_v)
        lax.fori_loop(jnp.int32(0), jnp.int32(V), vec_body, jnp.int32(0))

    pltpu.sync_copy(cnt_v, out_hbm.at[pl.ds(wid * jnp.int32(L), L)])
    pltpu.sync_copy(s_v, out_hbm.at[pl.ds(i32(NW * L) + wid * jnp.int32(L), L)])


def _combine_body(part_ref, out_ref):
    part = part_ref[...]
    cnt = jnp.sum(part[0], axis=0, keepdims=True)
    ssum = jnp.sum(part[1], axis=0, keepdims=True)
    lane = lax.broadcasted_iota(jnp.int32, (1, L), 1)
    valid = (lane < BINS) & (cnt > 0.0)
    nb = jnp.sum(valid.astype(jnp.float32))
    coeff = jnp.where(valid, 4.0 / jnp.where(valid, cnt, 1.0), 0.0)
    tot = jnp.sum(coeff * ssum)
    loss = jnp.where(nb > 0.0, -tot / jnp.maximum(nb, 1.0), 0.0)
    out_ref[...] = jnp.broadcast_to(loss, (1, 1))


def kernel(pred, target):
    pred_r = pred.reshape(B * C * P)
    tgt32 = target.astype(jnp.int32).reshape(B * P)
    partials = _ghm_partials(pred_r, tgt32).reshape(2, NW, L)
    loss2d = pl.pallas_call(
        _combine_body,
        out_shape=jax.ShapeDtypeStruct((1, 1), jnp.float32),
    )(partials)
    return loss2d[0, 0]


# speedup vs baseline: 5.5867x; 2.2236x over previous
import functools

import jax
import jax.numpy as jnp
import numpy as np
from jax import lax
from jax.experimental import pallas as pl
from jax.experimental.pallas import tpu as pltpu
from jax.experimental.pallas import tpu_sc as plsc

NC = 2
NS = 16
NW = NC * NS
L = 16
BINS = 10

B = 16
C = 3
H = 512
W = 512
P = H * W
PW = P // NW
ROWS = H // NW
V = PW // L
VPR = W // L

_EDGES = np.arange(BINS + 1, dtype=np.float32) / np.float32(10)
_EDGES[BINS] += np.float32(1e-6)
EDGES = [float(e) for e in _EDGES]

_LOG_C = [float(np.float32(2.0 / k)) for k in (11, 9, 7, 5, 3, 1)]

_EDGE_TAB = np.full((L,), np.inf, np.float32)
_EDGE_TAB[: BINS + 1] = _EDGES
UNROLL = 8

_mesh = plsc.VectorSubcoreMesh(
    core_axis_name="c", subcore_axis_name="s", num_cores=NC, num_subcores=NS
)


@functools.partial(
    pl.kernel,
    out_type=jax.ShapeDtypeStruct((2 * NW * L,), jnp.float32),
    mesh=_mesh,
    compiler_params=pltpu.CompilerParams(needs_layout_passes=False),
    scratch_types=[
        pltpu.VMEM((ROWS, W), jnp.float32),
        pltpu.VMEM((ROWS, W), jnp.float32),
        pltpu.VMEM((ROWS, W), jnp.float32),
        pltpu.VMEM((ROWS, W), jnp.int32),
        pltpu.VMEM((ROWS, W), jnp.float32),
        pltpu.VMEM((ROWS, W), jnp.float32),
        pltpu.VMEM((ROWS, W), jnp.float32),
        pltpu.VMEM((ROWS, W), jnp.int32),
        pltpu.SemaphoreType.DMA,
        pltpu.SemaphoreType.DMA,
        pltpu.VMEM((L,), jnp.float32),
        pltpu.VMEM((L,), jnp.float32),
        pltpu.VMEM((L,), jnp.float32),
        pltpu.VMEM((L,), jnp.float32),
        pltpu.VMEM((L,), jnp.float32),
        pltpu.VMEM((L,), jnp.float32),
        pltpu.VMEM((L,), jnp.float32),
        pltpu.VMEM((L,), jnp.float32),
        pltpu.VMEM((L,), jnp.float32),
    ],
)
def _ghm_partials(pred_hbm, tgt_hbm, edge_hbm, out_hbm, p0_v, p1_v, p2_v, t_v,
                  q0_v, q1_v, q2_v, u_v, sem_a, sem_b,
                  c0_v, c1_v, c2_v, c3_v, s0_v, s1_v, s2_v, s3_v, edge_v):
    wid = lax.axis_index("s") * NC + lax.axis_index("c")
    r0 = wid * jnp.int32(ROWS)
    cnt_refs = [c0_v, c1_v, c2_v, c3_v]
    s_refs = [s0_v, s1_v, s2_v, s3_v]
    for r in cnt_refs + s_refs:
        r[...] = jnp.zeros((L,), jnp.float32)
    pltpu.sync_copy(edge_hbm, edge_v)
    ones = jnp.ones((L,), jnp.float32)
    onei = jnp.ones((L,), jnp.int32)
    zeroi = jnp.zeros((L,), jnp.int32)

    iota16 = lax.iota(jnp.int32, L)

    def one_vec(bufs, r, cc, sl, cnt_v, s_v):
        p0 = bufs[0][r, sl]
        p1 = bufs[1][r, sl]
        p2 = bufs[2][r, sl]
        t = bufs[3][r, sl]
        m = jnp.maximum(jnp.maximum(p0, p1), p2)
        x0 = p0 - m
        x1 = p1 - m
        x2 = p2 - m
        e0 = jnp.exp(x0)
        e1 = jnp.exp(x1)
        e2 = jnp.exp(x2)
        s = e0 + e1 + e2
        is0 = t == 0
        is1 = t == 1
        xt = jnp.where(is0, x0, jnp.where(is1, x1, x2))
        et = jnp.where(is0, e0, jnp.where(is1, e1, e2))
        g = 1.0 - et / s
        z = (s - 1.0) / (s + 1.0)
        u = z * z
        poly = jnp.full((L,), _LOG_C[0], jnp.float32)
        for ck in _LOG_C[1:]:
            poly = poly * u + ck
        picked = xt - poly * z
        c = (g * 10.0).astype(jnp.int32)
        e_lo = plsc.load_gather(edge_v, [c])
        e_hi = plsc.load_gather(edge_v, [c + onei])
        bin_ = c + jnp.where(g >= e_hi, onei, zeroi) - jnp.where(g < e_lo, onei, zeroi)
        bin_ = jnp.maximum(bin_, zeroi)
        plsc.addupdate_scatter(cnt_v, [bin_], ones)
        plsc.addupdate_scatter(s_v, [bin_], picked)

    i32 = jnp.int32

    def compute(bufs):
        def vec_body(j):
            f0 = j * i32(UNROLL)
            for u in range(UNROLL):
                f = f0 + i32(u)
                r = lax.shift_right_logical(f, i32(5))
                cc = lax.shift_left(f & i32(VPR - 1), i32(4))
                one_vec(bufs, r, cc, pl.ds(cc, L), cnt_refs[u % 4], s_refs[u % 4])

        plsc.parallel_loop(jnp.int32(0), jnp.int32(V // UNROLL), jnp.int32(1))(vec_body)

    bufs_a = (p0_v, p1_v, p2_v, t_v)
    bufs_b = (q0_v, q1_v, q2_v, u_v)
    rsl = pl.ds(r0, ROWS)

    def fire(b, bufs, sem):
        pltpu.async_copy(pred_hbm.at[b, i32(0), rsl, :], bufs[0], sem)
        pltpu.async_copy(pred_hbm.at[b, i32(1), rsl, :], bufs[1], sem)
        pltpu.async_copy(pred_hbm.at[b, i32(2), rsl, :], bufs[2], sem)
        pltpu.async_copy(tgt_hbm.at[b, rsl, :], bufs[3], sem)

    def drain(bufs, sem):
        pltpu.make_async_copy(pred_hbm.at[i32(0), i32(0), rsl, :], bufs[0], sem).wait()
        pltpu.make_async_copy(pred_hbm.at[i32(0), i32(0), rsl, :], bufs[1], sem).wait()
        pltpu.make_async_copy(pred_hbm.at[i32(0), i32(0), rsl, :], bufs[2], sem).wait()
        pltpu.make_async_copy(tgt_hbm.at[i32(0), rsl, :], bufs[3], sem).wait()

    fire(i32(0), bufs_a, sem_a)

    def super_body(i, carry):
        b0 = i * i32(2)
        fire(b0 + i32(1), bufs_b, sem_b)
        drain(bufs_a, sem_a)
        compute(bufs_a)
        fire(jnp.minimum(b0 + i32(2), i32(B - 1)), bufs_a, sem_a)
        drain(bufs_b, sem_b)
        compute(bufs_b)
        return carry

    lax.fori_loop(jnp.int32(0), jnp.int32(B // 2), super_body, jnp.int32(0))
    drain(bufs_a, sem_a)

    c0_v[...] = (c0_v[...] + c1_v[...]) + (c2_v[...] + c3_v[...])
    s0_v[...] = (s0_v[...] + s1_v[...]) + (s2_v[...] + s3_v[...])
    pltpu.sync_copy(c0_v, out_hbm.at[pl.ds(wid * jnp.int32(L), L)])
    pltpu.sync_copy(s0_v, out_hbm.at[pl.ds(i32(NW * L) + wid * jnp.int32(L), L)])


def _combine_body(part_ref, out_ref):
    part = part_ref[...]
    cnt = jnp.sum(part[0], axis=0, keepdims=True)
    ssum = jnp.sum(part[1], axis=0, keepdims=True)
    lane = lax.broadcasted_iota(jnp.int32, (1, L), 1)
    valid = (lane < BINS) & (cnt > 0.0)
    nb = jnp.sum(valid.astype(jnp.float32))
    coeff = jnp.where(valid, 4.0 / jnp.where(valid, cnt, 1.0), 0.0)
    tot = jnp.sum(coeff * ssum)
    loss = jnp.where(nb > 0.0, -tot / jnp.maximum(nb, 1.0), 0.0)
    out_ref[...] = jnp.broadcast_to(loss, (1, 1))


def kernel(pred, target):
    tgt32 = target.astype(jnp.int32)
    edge_tab = jnp.asarray(_EDGE_TAB)
    partials = _ghm_partials(pred, tgt32, edge_tab).reshape(2, NW, L)
    loss2d = pl.pallas_call(
        _combine_body,
        out_shape=jax.ShapeDtypeStruct((1, 1), jnp.float32),
    )(partials)
    return loss2d[0, 0]


# speedup vs baseline: 5.5898x; 1.0006x over previous
import functools

import jax
import jax.numpy as jnp
import numpy as np
from jax import lax
from jax.experimental import pallas as pl
from jax.experimental.pallas import tpu as pltpu
from jax.experimental.pallas import tpu_sc as plsc

NC = 2
NS = 16
NW = NC * NS
L = 16
BINS = 10

B = 16
C = 3
H = 512
W = 512
P = H * W
PW = P // NW
ROWS = H // NW
V = PW // L
VPR = W // L

_EDGES = np.arange(BINS + 1, dtype=np.float32) / np.float32(10)
_EDGES[BINS] += np.float32(1e-6)
EDGES = [float(e) for e in _EDGES]

_LOG_C = [float(np.float32(2.0 / k)) for k in (11, 9, 7, 5, 3, 1)]

_EDGE_TAB = np.full((L,), np.inf, np.float32)
_EDGE_TAB[: BINS + 1] = _EDGES
UNROLL = 4

_mesh = plsc.VectorSubcoreMesh(
    core_axis_name="c", subcore_axis_name="s", num_cores=NC, num_subcores=NS
)


@functools.partial(
    pl.kernel,
    out_type=jax.ShapeDtypeStruct((2 * NW * L,), jnp.float32),
    mesh=_mesh,
    compiler_params=pltpu.CompilerParams(needs_layout_passes=False),
    scratch_types=[
        pltpu.VMEM((ROWS, W), jnp.float32),
        pltpu.VMEM((ROWS, W), jnp.float32),
        pltpu.VMEM((ROWS, W), jnp.float32),
        pltpu.VMEM((ROWS, W), jnp.int32),
        pltpu.VMEM((ROWS, W), jnp.float32),
        pltpu.VMEM((ROWS, W), jnp.float32),
        pltpu.VMEM((ROWS, W), jnp.float32),
        pltpu.VMEM((ROWS, W), jnp.int32),
        pltpu.SemaphoreType.DMA,
        pltpu.SemaphoreType.DMA,
        pltpu.VMEM((L,), jnp.float32),
        pltpu.VMEM((L,), jnp.float32),
        pltpu.VMEM((L,), jnp.float32),
        pltpu.VMEM((L,), jnp.float32),
        pltpu.VMEM((L,), jnp.float32),
        pltpu.VMEM((L,), jnp.float32),
        pltpu.VMEM((L,), jnp.float32),
        pltpu.VMEM((L,), jnp.float32),
        pltpu.VMEM((L,), jnp.float32),
    ],
)
def _ghm_partials(pred_hbm, tgt_hbm, edge_hbm, out_hbm, p0_v, p1_v, p2_v, t_v,
                  q0_v, q1_v, q2_v, u_v, sem_a, sem_b,
                  c0_v, c1_v, c2_v, c3_v, s0_v, s1_v, s2_v, s3_v, edge_v):
    wid = lax.axis_index("s") * NC + lax.axis_index("c")
    r0 = wid * jnp.int32(ROWS)
    cnt_refs = [c0_v, c1_v, c2_v, c3_v]
    s_refs = [s0_v, s1_v, s2_v, s3_v]
    for r in cnt_refs + s_refs:
        r[...] = jnp.zeros((L,), jnp.float32)
    pltpu.sync_copy(edge_hbm, edge_v)
    ones = jnp.ones((L,), jnp.float32)
    onei = jnp.ones((L,), jnp.int32)
    zeroi = jnp.zeros((L,), jnp.int32)

    iota16 = lax.iota(jnp.int32, L)

    def one_vec(bufs, r, cc, sl, cnt_v, s_v):
        p0 = bufs[0][r, sl]
        p1 = bufs[1][r, sl]
        p2 = bufs[2][r, sl]
        t = bufs[3][r, sl]
        m = jnp.maximum(jnp.maximum(p0, p1), p2)
        x0 = p0 - m
        x1 = p1 - m
        x2 = p2 - m
        e0 = jnp.exp(x0)
        e1 = jnp.exp(x1)
        e2 = jnp.exp(x2)
        s = e0 + e1 + e2
        is0 = t == 0
        is1 = t == 1
        xt = jnp.where(is0, x0, jnp.where(is1, x1, x2))
        et = jnp.where(is0, e0, jnp.where(is1, e1, e2))
        g = 1.0 - et / s
        z = (s - 1.0) / (s + 1.0)
        u = z * z
        poly = jnp.full((L,), _LOG_C[0], jnp.float32)
        for ck in _LOG_C[1:]:
            poly = poly * u + ck
        picked = xt - poly * z
        c = (g * 10.0).astype(jnp.int32)
        e_lo = plsc.load_gather(edge_v, [c])
        e_hi = plsc.load_gather(edge_v, [c + onei])
        bin_ = c + jnp.where(g >= e_hi, onei, zeroi) - jnp.where(g < e_lo, onei, zeroi)
        bin_ = jnp.maximum(bin_, zeroi)
        plsc.addupdate_scatter(cnt_v, [bin_], ones)
        plsc.addupdate_scatter(s_v, [bin_], picked)

    i32 = jnp.int32

    def compute(bufs):
        def vec_body(j):
            f0 = j * i32(UNROLL)
            for u in range(UNROLL):
                f = f0 + i32(u)
                r = lax.shift_right_logical(f, i32(5))
                cc = lax.shift_left(f & i32(VPR - 1), i32(4))
                one_vec(bufs, r, cc, pl.ds(cc, L), cnt_refs[u], s_refs[u])

        plsc.parallel_loop(jnp.int32(0), jnp.int32(V // UNROLL), jnp.int32(1))(vec_body)

    bufs_a = (p0_v, p1_v, p2_v, t_v)
    bufs_b = (q0_v, q1_v, q2_v, u_v)
    rsl = pl.ds(r0, ROWS)

    def fire(b, bufs, sem):
        pltpu.async_copy(pred_hbm.at[b, i32(0), rsl, :], bufs[0], sem)
        pltpu.async_copy(pred_hbm.at[b, i32(1), rsl, :], bufs[1], sem)
        pltpu.async_copy(pred_hbm.at[b, i32(2), rsl, :], bufs[2], sem)
        pltpu.async_copy(tgt_hbm.at[b, rsl, :], bufs[3], sem)

    def drain(bufs, sem):
        pltpu.make_async_copy(pred_hbm.at[i32(0), i32(0), rsl, :], bufs[0], sem).wait()
        pltpu.make_async_copy(pred_hbm.at[i32(0), i32(0), rsl, :], bufs[1], sem).wait()
        pltpu.make_async_copy(pred_hbm.at[i32(0), i32(0), rsl, :], bufs[2], sem).wait()
        pltpu.make_async_copy(tgt_hbm.at[i32(0), rsl, :], bufs[3], sem).wait()

    fire(i32(0), bufs_a, sem_a)

    def super_body(i, carry):
        b0 = i * i32(2)
        fire(b0 + i32(1), bufs_b, sem_b)
        drain(bufs_a, sem_a)
        compute(bufs_a)
        fire(jnp.minimum(b0 + i32(2), i32(B - 1)), bufs_a, sem_a)
        drain(bufs_b, sem_b)
        compute(bufs_b)
        return carry

    lax.fori_loop(jnp.int32(0), jnp.int32(B // 2), super_body, jnp.int32(0))
    drain(bufs_a, sem_a)

    c0_v[...] = (c0_v[...] + c1_v[...]) + (c2_v[...] + c3_v[...])
    s0_v[...] = (s0_v[...] + s1_v[...]) + (s2_v[...] + s3_v[...])
    pltpu.sync_copy(c0_v, out_hbm.at[pl.ds(wid * jnp.int32(L), L)])
    pltpu.sync_copy(s0_v, out_hbm.at[pl.ds(i32(NW * L) + wid * jnp.int32(L), L)])


def _combine_body(part_ref, out_ref):
    part = part_ref[...]
    cnt = jnp.sum(part[0], axis=0, keepdims=True)
    ssum = jnp.sum(part[1], axis=0, keepdims=True)
    lane = lax.broadcasted_iota(jnp.int32, (1, L), 1)
    valid = (lane < BINS) & (cnt > 0.0)
    nb = jnp.sum(valid.astype(jnp.float32))
    coeff = jnp.where(valid, 4.0 / jnp.where(valid, cnt, 1.0), 0.0)
    tot = jnp.sum(coeff * ssum)
    loss = jnp.where(nb > 0.0, -tot / jnp.maximum(nb, 1.0), 0.0)
    out_ref[...] = jnp.broadcast_to(loss, (1, 1))


def kernel(pred, target):
    tgt32 = target.astype(jnp.int32)
    edge_tab = jnp.asarray(_EDGE_TAB)
    partials = _ghm_partials(pred, tgt32, edge_tab).reshape(2, NW, L)
    loss2d = pl.pallas_call(
        _combine_body,
        out_shape=jax.ShapeDtypeStruct((1, 1), jnp.float32),
    )(partials)
    return loss2d[0, 0]
